# initial kernel scaffold (unmeasured)
import jax
import jax.numpy as jnp
from jax import lax
from jax.experimental import pallas as pl
from jax.experimental.pallas import tpu as pltpu

E = 8
E_LOC = 4
C = 320


def _moe_body(x_mine_ref, x_rem_ref, w1_ref, w2_ref,
              out_local_ref, out_recv_ref,
              recv_buf, yback_buf, res_buf,
              send1, recv1, send2, recv2):
    my_x = lax.axis_index("x")
    my_y = lax.axis_index("y")
    my_z = lax.axis_index("z")
    peer = (my_x, 1 - my_y, my_z)

    barrier = pltpu.get_barrier_semaphore()
    pl.semaphore_signal(barrier, inc=1, device_id=peer,
                        device_id_type=pl.DeviceIdType.MESH)
    pl.semaphore_wait(barrier, 1)

    rdma1 = pltpu.make_async_remote_copy(
        src_ref=x_rem_ref, dst_ref=recv_buf,
        send_sem=send1, recv_sem=recv1,
        device_id=peer, device_id_type=pl.DeviceIdType.MESH)
    rdma1.start()

    for e in range(E_LOC):
        h = jnp.maximum(
            jnp.dot(x_mine_ref[e], w1_ref[e],
                    preferred_element_type=jnp.float32), 0.0)
        out_local_ref[e] = jnp.dot(h.astype(jnp.bfloat16), w2_ref[e],
                                   preferred_element_type=jnp.float32)

    rdma1.wait_send()
    rdma1.wait_recv()

    for e in range(E_LOC):
        h = jnp.maximum(
            jnp.dot(recv_buf[e], w1_ref[e],
                    preferred_element_type=jnp.float32), 0.0)
        yback_buf[e] = jnp.dot(h.astype(jnp.bfloat16), w2_ref[e],
                               preferred_element_type=jnp.float32
                               ).astype(jnp.bfloat16)

    rdma2 = pltpu.make_async_remote_copy(
        src_ref=yback_buf, dst_ref=res_buf,
        send_sem=send2, recv_sem=recv2,
        device_id=peer, device_id_type=pl.DeviceIdType.MESH)
    rdma2.start()
    rdma2.wait_send()
    rdma2.wait_recv()

    out_recv_ref[...] = res_buf[...].astype(jnp.float32)


def kernel(x, assign, W1, W2):
    T, D = x.shape
    my_y = lax.axis_index("y")

    xb = x.astype(jnp.bfloat16)
    w1b = W1.astype(jnp.bfloat16)
    w2b = W2.astype(jnp.bfloat16)

    order = jnp.argsort(assign, stable=True).astype(jnp.int32)
    sorted_assign = assign[order]
    counts = jnp.zeros((E,), jnp.int32).at[assign].add(1)
    offsets = jnp.cumsum(counts) - counts
    ranks = jnp.arange(T, dtype=jnp.int32) - offsets[sorted_assign]
    idx_map = jnp.full((E, C), T, jnp.int32).at[sorted_assign, ranks].set(
        order, mode="drop")

    x_pad = jnp.concatenate([xb, jnp.zeros((1, D), jnp.bfloat16)], axis=0)
    x_routed = x_pad[idx_map]

    e0_mine = my_y * E_LOC
    e0_rem = (1 - my_y) * E_LOC
    x_mine = lax.dynamic_slice_in_dim(x_routed, e0_mine, E_LOC, axis=0)
    x_rem = lax.dynamic_slice_in_dim(x_routed, e0_rem, E_LOC, axis=0)

    out_local, out_recv = pl.pallas_call(
        _moe_body,
        out_shape=(
            jax.ShapeDtypeStruct((E_LOC, C, D), jnp.float32),
            jax.ShapeDtypeStruct((E_LOC, C, D), jnp.float32),
        ),
        in_specs=[pl.BlockSpec(memory_space=pltpu.VMEM)] * 4,
        out_specs=(pl.BlockSpec(memory_space=pltpu.VMEM),
                   pl.BlockSpec(memory_space=pltpu.VMEM)),
        scratch_shapes=[
            pltpu.VMEM((E_LOC, C, D), jnp.bfloat16),
            pltpu.VMEM((E_LOC, C, D), jnp.bfloat16),
            pltpu.VMEM((E_LOC, C, D), jnp.bfloat16),
            pltpu.SemaphoreType.DMA,
            pltpu.SemaphoreType.DMA,
            pltpu.SemaphoreType.DMA,
            pltpu.SemaphoreType.DMA,
        ],
        compiler_params=pltpu.CompilerParams(collective_id=0),
    )(x_mine, x_rem, w1b, w2b)

    results = jnp.zeros((E, C, D), jnp.float32)
    results = lax.dynamic_update_slice(results, out_local, (e0_mine, 0, 0))
    results = lax.dynamic_update_slice(results, out_recv, (e0_rem, 0, 0))

    out = jnp.zeros((T, D), jnp.float32).at[idx_map.reshape(-1)].set(
        results.reshape(E * C, D), mode="drop")
    return out


# baseline (device time: 197379 ns/iter reference)
import jax
import jax.numpy as jnp
from jax import lax
from jax.experimental import pallas as pl
from jax.experimental.pallas import tpu as pltpu

E = 8
E_LOC = 4
C = 320


def _moe_body(x_mine_ref, x_rem_ref, w1_ref, w2_ref,
              out_local_ref, out_recv_ref,
              recv_buf, yback_buf, res_buf,
              send1, recv1, send2, recv2):
    my_x = lax.axis_index("x")
    my_y = lax.axis_index("y")
    my_z = lax.axis_index("z")
    peer = (my_x, 1 - my_y, my_z)

    barrier = pltpu.get_barrier_semaphore()
    pl.semaphore_signal(barrier, inc=1, device_id=peer,
                        device_id_type=pl.DeviceIdType.MESH)
    pl.semaphore_wait(barrier, 1)

    rdma1 = pltpu.make_async_remote_copy(
        src_ref=x_rem_ref, dst_ref=recv_buf,
        send_sem=send1, recv_sem=recv1,
        device_id=peer, device_id_type=pl.DeviceIdType.MESH)
    rdma1.start()

    for e in range(E_LOC):
        h = jnp.maximum(
            jnp.dot(x_mine_ref[e], w1_ref[e],
                    preferred_element_type=jnp.float32), 0.0)
        out_local_ref[e] = jnp.dot(h.astype(jnp.bfloat16), w2_ref[e],
                                   preferred_element_type=jnp.float32)

    rdma1.wait_send()
    rdma1.wait_recv()

    for e in range(E_LOC):
        h = jnp.maximum(
            jnp.dot(recv_buf[e], w1_ref[e],
                    preferred_element_type=jnp.float32), 0.0)
        yback_buf[e] = jnp.dot(h.astype(jnp.bfloat16), w2_ref[e],
                               preferred_element_type=jnp.float32
                               ).astype(jnp.bfloat16)

    rdma2 = pltpu.make_async_remote_copy(
        src_ref=yback_buf, dst_ref=res_buf,
        send_sem=send2, recv_sem=recv2,
        device_id=peer, device_id_type=pl.DeviceIdType.MESH)
    rdma2.start()
    rdma2.wait_send()
    rdma2.wait_recv()

    out_recv_ref[...] = res_buf[...].astype(jnp.float32)


def kernel(x, assign, W1, W2):
    T, D = x.shape
    my_y = lax.axis_index("y")

    xb = x.astype(jnp.bfloat16)
    w1b = W1.astype(jnp.bfloat16)
    w2b = W2.astype(jnp.bfloat16)

    order = jnp.argsort(assign, stable=True).astype(jnp.int32)
    sorted_assign = assign[order]
    counts = jnp.zeros((E,), jnp.int32).at[assign].add(1)
    offsets = jnp.cumsum(counts) - counts
    ranks = jnp.arange(T, dtype=jnp.int32) - offsets[sorted_assign]
    idx_map = jnp.full((E, C), T, jnp.int32).at[sorted_assign, ranks].set(
        order, mode="drop")

    x_pad = jnp.concatenate([xb, jnp.zeros((1, D), jnp.bfloat16)], axis=0)
    x_routed = x_pad[idx_map]

    e0_mine = my_y * E_LOC
    e0_rem = (1 - my_y) * E_LOC
    x_mine = lax.dynamic_slice_in_dim(x_routed, e0_mine, E_LOC, axis=0)
    x_rem = lax.dynamic_slice_in_dim(x_routed, e0_rem, E_LOC, axis=0)

    out_local, out_recv = pl.pallas_call(
        _moe_body,
        out_shape=(
            jax.ShapeDtypeStruct((E_LOC, C, D), jnp.float32),
            jax.ShapeDtypeStruct((E_LOC, C, D), jnp.float32),
        ),
        in_specs=[pl.BlockSpec(memory_space=pltpu.VMEM)] * 4,
        out_specs=(pl.BlockSpec(memory_space=pltpu.VMEM),
                   pl.BlockSpec(memory_space=pltpu.VMEM)),
        scratch_shapes=[
            pltpu.VMEM((E_LOC, C, D), jnp.bfloat16),
            pltpu.VMEM((E_LOC, C, D), jnp.bfloat16),
            pltpu.VMEM((E_LOC, C, D), jnp.bfloat16),
            pltpu.SemaphoreType.DMA,
            pltpu.SemaphoreType.DMA,
            pltpu.SemaphoreType.DMA,
            pltpu.SemaphoreType.DMA,
        ],
        compiler_params=pltpu.CompilerParams(
            collective_id=0, vmem_limit_bytes=100 * 1024 * 1024),
    )(x_mine, x_rem, w1b, w2b)

    results = jnp.zeros((E, C, D), jnp.float32)
    results = lax.dynamic_update_slice(results, out_local, (e0_mine, 0, 0))
    results = lax.dynamic_update_slice(results, out_recv, (e0_rem, 0, 0))

    out = jnp.zeros((T, D), jnp.float32).at[idx_map.reshape(-1)].set(
        results.reshape(E * C, D), mode="drop")
    return out


# device time: 141221 ns/iter; 1.3977x vs baseline; 1.3977x over previous
import jax
import jax.numpy as jnp
from jax import lax
from jax.experimental import pallas as pl
from jax.experimental.pallas import tpu as pltpu

E = 8
E_LOC = 4
C = 320


def _moe_body(xb_ref, tokm_ref, tokr_ref, rank_ref, asg_ref, w1_ref, w2_ref,
              out_ref, xrem_buf, recv_buf, yback_buf, res_buf,
              w1_vmem, w2_vmem,
              send1, recv1, send2, recv2, wsem1, wsem2):
    T, D = xb_ref.shape
    my_x = lax.axis_index("x")
    my_y = lax.axis_index("y")
    my_z = lax.axis_index("z")
    peer = (my_x, 1 - my_y, my_z)
    e0m = my_y * E_LOC
    e0r = (1 - my_y) * E_LOC

    barrier = pltpu.get_barrier_semaphore()
    pl.semaphore_signal(barrier, inc=1, device_id=peer,
                        device_id_type=pl.DeviceIdType.MESH)
    pl.semaphore_wait(barrier, 1)

    def w_copies(e, slot):
        return (pltpu.make_async_copy(w1_ref.at[e], w1_vmem.at[slot],
                                      wsem1.at[slot]),
                pltpu.make_async_copy(w2_ref.at[e], w2_vmem.at[slot],
                                      wsem2.at[slot]))

    for c in w_copies(0, 0):
        c.start()

    iota_ct = lax.broadcasted_iota(jnp.int32, (C, T), 1)
    iota_tc = lax.broadcasted_iota(jnp.int32, (T, C), 1)
    xb = xb_ref[...]

    def chunk_rdma(e, src, dst, ssem, rsem):
        return pltpu.make_async_remote_copy(
            src_ref=src.at[e], dst_ref=dst.at[e],
            send_sem=ssem.at[e], recv_sem=rsem.at[e],
            device_id=peer, device_id_type=pl.DeviceIdType.MESH)

    rdma1 = []
    for e in range(E_LOC):
        p = (tokr_ref[e] == iota_ct).astype(jnp.bfloat16)
        xrem_buf[e] = jnp.dot(
            p, xb, preferred_element_type=jnp.float32).astype(jnp.bfloat16)
        r = chunk_rdma(e, xrem_buf, recv_buf, send1, recv1)
        r.start()
        rdma1.append(r)

    out_ref[...] = jnp.zeros((T, D), jnp.float32)

    rdma2 = []
    for e in range(E_LOC):
        slot = e % 2
        for c in w_copies(e, slot):
            c.wait()
        if e + 1 < E_LOC:
            for c in w_copies(e + 1, (e + 1) % 2):
                c.start()
        w1e = w1_vmem[slot]
        w2e = w2_vmem[slot]
        pm = (tokm_ref[e] == iota_ct).astype(jnp.bfloat16)
        xe = jnp.dot(pm, xb, preferred_element_type=jnp.float32
                     ).astype(jnp.bfloat16)
        h = jnp.maximum(jnp.dot(xe, w1e,
                                preferred_element_type=jnp.float32), 0.0)
        y = jnp.dot(h.astype(jnp.bfloat16), w2e,
                    preferred_element_type=jnp.float32)
        qm = ((asg_ref[...] == e0m + e) &
              (rank_ref[...] == iota_tc)).astype(jnp.bfloat16)
        out_ref[...] += jnp.dot(qm, y.astype(jnp.bfloat16),
                                preferred_element_type=jnp.float32)

        rdma1[e].wait_send()
        rdma1[e].wait_recv()
        h2 = jnp.maximum(jnp.dot(recv_buf[e], w1e,
                                 preferred_element_type=jnp.float32), 0.0)
        yback_buf[e] = jnp.dot(h2.astype(jnp.bfloat16), w2e,
                               preferred_element_type=jnp.float32
                               ).astype(jnp.bfloat16)
        r = chunk_rdma(e, yback_buf, res_buf, send2, recv2)
        r.start()
        rdma2.append(r)

    for e in range(E_LOC):
        rdma2[e].wait_send()
        rdma2[e].wait_recv()
        qr = ((asg_ref[...] == e0r + e) &
              (rank_ref[...] == iota_tc)).astype(jnp.bfloat16)
        out_ref[...] += jnp.dot(qr, res_buf[e],
                                preferred_element_type=jnp.float32)


def kernel(x, assign, W1, W2):
    T, D = x.shape
    F = W1.shape[2]
    my_y = lax.axis_index("y")

    xb = x.astype(jnp.bfloat16)
    w1b = W1.astype(jnp.bfloat16)
    w2b = W2.astype(jnp.bfloat16)

    onehot = (assign[:, None] == jnp.arange(E, dtype=assign.dtype)[None, :]
              ).astype(jnp.int32)
    excl = jnp.cumsum(onehot, axis=0) - onehot
    rank = excl[jnp.arange(T), assign]
    tok_of_slot = jnp.full((E, C), T, jnp.int32).at[assign, rank].set(
        jnp.arange(T, dtype=jnp.int32), mode="drop")

    tokm = lax.dynamic_slice_in_dim(tok_of_slot, my_y * E_LOC,
                                    E_LOC, axis=0)[..., None]
    tokr = lax.dynamic_slice_in_dim(tok_of_slot, (1 - my_y) * E_LOC,
                                    E_LOC, axis=0)[..., None]
    rank2 = rank.astype(jnp.int32)[:, None]
    asg2 = assign.astype(jnp.int32)[:, None]

    return pl.pallas_call(
        _moe_body,
        out_shape=jax.ShapeDtypeStruct((T, D), jnp.float32),
        in_specs=[pl.BlockSpec(memory_space=pltpu.VMEM)] * 5
        + [pl.BlockSpec(memory_space=pl.ANY)] * 2,
        out_specs=pl.BlockSpec(memory_space=pltpu.VMEM),
        scratch_shapes=[
            pltpu.VMEM((E_LOC, C, D), jnp.bfloat16),
            pltpu.VMEM((E_LOC, C, D), jnp.bfloat16),
            pltpu.VMEM((E_LOC, C, D), jnp.bfloat16),
            pltpu.VMEM((E_LOC, C, D), jnp.bfloat16),
            pltpu.VMEM((2, D, F), jnp.bfloat16),
            pltpu.VMEM((2, F, D), jnp.bfloat16),
            pltpu.SemaphoreType.DMA((E_LOC,)),
            pltpu.SemaphoreType.DMA((E_LOC,)),
            pltpu.SemaphoreType.DMA((E_LOC,)),
            pltpu.SemaphoreType.DMA((E_LOC,)),
            pltpu.SemaphoreType.DMA((2,)),
            pltpu.SemaphoreType.DMA((2,)),
        ],
        compiler_params=pltpu.CompilerParams(
            collective_id=0, vmem_limit_bytes=100 * 1024 * 1024),
    )(xb, tokm, tokr, rank2, asg2, w1b, w2b)


# device time: 124111 ns/iter; 1.5903x vs baseline; 1.1379x over previous
import jax
import jax.numpy as jnp
from jax import lax
from jax.experimental import pallas as pl
from jax.experimental.pallas import tpu as pltpu

E = 8
E_LOC = 4
C = 320


def _moe_body(xb_ref, asg_1t_ref, rank_1t_ref, asg_t1_ref, rank_t1_ref,
              w1_ref, w2_ref,
              out_ref, xrem_buf, recv_buf, yback_buf, res_buf,
              w1_vmem, w2_vmem,
              send1, recv1, send2, recv2, wsem1, wsem2):
    T, D = xb_ref.shape
    my_x = lax.axis_index("x")
    my_y = lax.axis_index("y")
    my_z = lax.axis_index("z")
    peer = (my_x, 1 - my_y, my_z)
    e0m = my_y * E_LOC
    e0r = (1 - my_y) * E_LOC

    barrier = pltpu.get_barrier_semaphore()
    pl.semaphore_signal(barrier, inc=1, device_id=peer,
                        device_id_type=pl.DeviceIdType.MESH)
    pl.semaphore_wait(barrier, 1)

    def w_copies(e, slot):
        return (pltpu.make_async_copy(w1_ref.at[e], w1_vmem.at[slot],
                                      wsem1.at[slot]),
                pltpu.make_async_copy(w2_ref.at[e], w2_vmem.at[slot],
                                      wsem2.at[slot]))

    for c in w_copies(0, 0):
        c.start()

    iota_c_ct = lax.broadcasted_iota(jnp.int32, (C, T), 0)
    iota_c_tc = lax.broadcasted_iota(jnp.int32, (T, C), 1)
    asg_1t = asg_1t_ref[...]
    rank_1t = rank_1t_ref[...]
    asg_t1 = asg_t1_ref[...]
    rank_t1 = rank_t1_ref[...]
    xb = xb_ref[...]

    def p_mat(e_glob):
        return ((asg_1t == e_glob) &
                (rank_1t == iota_c_ct)).astype(jnp.bfloat16)

    def q_mat(e_glob):
        return ((asg_t1 == e_glob) &
                (rank_t1 == iota_c_tc)).astype(jnp.bfloat16)

    def chunk_rdma(e, src, dst, ssem, rsem):
        return pltpu.make_async_remote_copy(
            src_ref=src.at[e], dst_ref=dst.at[e],
            send_sem=ssem.at[e], recv_sem=rsem.at[e],
            device_id=peer, device_id_type=pl.DeviceIdType.MESH)

    rdma1 = []
    for e in range(E_LOC):
        xrem_buf[e] = jnp.dot(
            p_mat(e0r + e), xb,
            preferred_element_type=jnp.float32).astype(jnp.bfloat16)
        r = chunk_rdma(e, xrem_buf, recv_buf, send1, recv1)
        r.start()
        rdma1.append(r)

    out_ref[...] = jnp.zeros((T, D), jnp.float32)

    rdma2 = []
    for e in range(E_LOC):
        slot = e % 2
        for c in w_copies(e, slot):
            c.wait()
        if e + 1 < E_LOC:
            for c in w_copies(e + 1, (e + 1) % 2):
                c.start()
        w1e = w1_vmem[slot]
        w2e = w2_vmem[slot]
        xe = jnp.dot(p_mat(e0m + e), xb,
                     preferred_element_type=jnp.float32
                     ).astype(jnp.bfloat16)
        h = jnp.maximum(jnp.dot(xe, w1e,
                                preferred_element_type=jnp.float32), 0.0)
        y = jnp.dot(h.astype(jnp.bfloat16), w2e,
                    preferred_element_type=jnp.float32)
        out_ref[...] += jnp.dot(q_mat(e0m + e), y.astype(jnp.bfloat16),
                                preferred_element_type=jnp.float32)

        rdma1[e].wait_send()
        rdma1[e].wait_recv()
        h2 = jnp.maximum(jnp.dot(recv_buf[e], w1e,
                                 preferred_element_type=jnp.float32), 0.0)
        yback_buf[e] = jnp.dot(h2.astype(jnp.bfloat16), w2e,
                               preferred_element_type=jnp.float32
                               ).astype(jnp.bfloat16)
        r = chunk_rdma(e, yback_buf, res_buf, send2, recv2)
        r.start()
        rdma2.append(r)

    for e in range(E_LOC):
        rdma2[e].wait_send()
        rdma2[e].wait_recv()
        out_ref[...] += jnp.dot(q_mat(e0r + e), res_buf[e],
                                preferred_element_type=jnp.float32)


def kernel(x, assign, W1, W2):
    T, D = x.shape
    F = W1.shape[2]

    xb = x.astype(jnp.bfloat16)
    w1b = W1.astype(jnp.bfloat16)
    w2b = W2.astype(jnp.bfloat16)

    asg = assign.astype(jnp.int32)
    onehot = (asg[:, None] == jnp.arange(E, dtype=jnp.int32)[None, :]
              ).astype(jnp.int32)
    excl = jnp.cumsum(onehot, axis=0) - onehot
    rank = jnp.sum(excl * onehot, axis=1).astype(jnp.int32)

    return pl.pallas_call(
        _moe_body,
        out_shape=jax.ShapeDtypeStruct((T, D), jnp.float32),
        in_specs=[pl.BlockSpec(memory_space=pltpu.VMEM)] * 5
        + [pl.BlockSpec(memory_space=pl.ANY)] * 2,
        out_specs=pl.BlockSpec(memory_space=pltpu.VMEM),
        scratch_shapes=[
            pltpu.VMEM((E_LOC, C, D), jnp.bfloat16),
            pltpu.VMEM((E_LOC, C, D), jnp.bfloat16),
            pltpu.VMEM((E_LOC, C, D), jnp.bfloat16),
            pltpu.VMEM((E_LOC, C, D), jnp.bfloat16),
            pltpu.VMEM((2, D, F), jnp.bfloat16),
            pltpu.VMEM((2, F, D), jnp.bfloat16),
            pltpu.SemaphoreType.DMA((E_LOC,)),
            pltpu.SemaphoreType.DMA((E_LOC,)),
            pltpu.SemaphoreType.DMA((E_LOC,)),
            pltpu.SemaphoreType.DMA((E_LOC,)),
            pltpu.SemaphoreType.DMA((2,)),
            pltpu.SemaphoreType.DMA((2,)),
        ],
        compiler_params=pltpu.CompilerParams(
            collective_id=0, vmem_limit_bytes=100 * 1024 * 1024),
    )(xb, asg[None, :], rank[None, :], asg[:, None], rank[:, None],
      w1b, w2b)


# device time: 91329 ns/iter; 2.1612x vs baseline; 1.3589x over previous
import jax
import jax.numpy as jnp
from jax import lax
from jax.experimental import pallas as pl
from jax.experimental.pallas import tpu as pltpu

E = 8
E_LOC = 4
C = 320


def _moe_body(xb_ref, asg_ref, rank_ref, w1_ref, w2_ref,
              out_ref, xrem_buf, yback_buf, recv_buf, res_buf,
              w1f, w2f,
              send1, recv1, send2, recv2, wsem1, wsem2):
    T, D = xb_ref.shape
    my_x = lax.axis_index("x")
    my_y = lax.axis_index("y")
    my_z = lax.axis_index("z")
    peer = (my_x, 1 - my_y, my_z)
    e0m = my_y * E_LOC
    e0r = (1 - my_y) * E_LOC

    barrier = pltpu.get_barrier_semaphore()
    pl.semaphore_signal(barrier, inc=1, device_id=peer,
                        device_id_type=pl.DeviceIdType.MESH)
    pl.semaphore_wait(barrier, 1)

    def w_copies(e):
        return (pltpu.make_async_copy(w1_ref.at[e], w1f, wsem1),
                pltpu.make_async_copy(w2_ref.at[e], w2f, wsem2))

    for c in w_copies(0):
        c.start()

    iota_c = lax.broadcasted_iota(jnp.int32, (C, T), 0)
    asg = asg_ref[...]
    rank = rank_ref[...]
    xb = xb_ref[...]

    def p_mat(e_glob):
        return ((asg == e_glob) & (rank == iota_c)).astype(jnp.bfloat16)

    def pt_dot(p, y):
        return lax.dot_general(p, y, (((0,), (0,)), ((), ())),
                               preferred_element_type=jnp.float32)

    def chunk_rdma(e, src, dst, ssem, rsem):
        return pltpu.make_async_remote_copy(
            src_ref=src.at[e], dst_ref=dst.at[e],
            send_sem=ssem.at[e], recv_sem=rsem.at[e],
            device_id=peer, device_id_type=pl.DeviceIdType.MESH)

    rdma1 = []
    for e in range(E_LOC):
        xrem_buf[e] = jnp.dot(
            p_mat(e0r + e), xb,
            preferred_element_type=jnp.float32).astype(jnp.bfloat16)
        r = chunk_rdma(e, xrem_buf, recv_buf, send1, recv1)
        r.start()
        rdma1.append(r)

    out_ref[...] = jnp.zeros((T, D), jnp.bfloat16)

    rdma2 = []
    for e in range(E_LOC):
        for c in w_copies(e):
            c.wait()
        w1e = w1f[...].astype(jnp.bfloat16)
        w2e = w2f[...].astype(jnp.bfloat16)
        if e + 1 < E_LOC:
            for c in w_copies(e + 1):
                c.start()
        pm = p_mat(e0m + e)
        xe = jnp.dot(pm, xb, preferred_element_type=jnp.float32
                     ).astype(jnp.bfloat16)
        h = jnp.maximum(jnp.dot(xe, w1e,
                                preferred_element_type=jnp.float32), 0.0)
        y = jnp.dot(h.astype(jnp.bfloat16), w2e,
                    preferred_element_type=jnp.float32)
        out_ref[...] += pt_dot(pm, y.astype(jnp.bfloat16)
                               ).astype(jnp.bfloat16)

        rdma1[e].wait_send()
        rdma1[e].wait_recv()
        h2 = jnp.maximum(jnp.dot(recv_buf[e], w1e,
                                 preferred_element_type=jnp.float32), 0.0)
        yback_buf[e] = jnp.dot(h2.astype(jnp.bfloat16), w2e,
                               preferred_element_type=jnp.float32
                               ).astype(jnp.bfloat16)
        r = chunk_rdma(e, yback_buf, res_buf, send2, recv2)
        r.start()
        rdma2.append(r)

    for e in range(E_LOC):
        rdma2[e].wait_send()
        rdma2[e].wait_recv()
        out_ref[...] += pt_dot(p_mat(e0r + e), res_buf[e]
                               ).astype(jnp.bfloat16)


def kernel(x, assign, W1, W2):
    T, D = x.shape
    F = W1.shape[2]

    xb = x.astype(jnp.bfloat16)

    asg = assign.astype(jnp.int32)
    onehot = (asg[:, None] == jnp.arange(E, dtype=jnp.int32)[None, :]
              ).astype(jnp.int32)
    excl = jnp.cumsum(onehot, axis=0) - onehot
    rank = jnp.sum(excl * onehot, axis=1).astype(jnp.int32)

    out = pl.pallas_call(
        _moe_body,
        out_shape=jax.ShapeDtypeStruct((T, D), jnp.bfloat16),
        in_specs=[pl.BlockSpec(memory_space=pltpu.VMEM)] * 3
        + [pl.BlockSpec(memory_space=pl.ANY)] * 2,
        out_specs=pl.BlockSpec(memory_space=pltpu.VMEM),
        scratch_shapes=[
            pltpu.VMEM((E_LOC, C, D), jnp.bfloat16),
            pltpu.VMEM((E_LOC, C, D), jnp.bfloat16),
            pltpu.VMEM((E_LOC, C, D), jnp.bfloat16),
            pltpu.VMEM((E_LOC, C, D), jnp.bfloat16),
            pltpu.VMEM((D, F), jnp.float32),
            pltpu.VMEM((F, D), jnp.float32),
            pltpu.SemaphoreType.DMA((E_LOC,)),
            pltpu.SemaphoreType.DMA((E_LOC,)),
            pltpu.SemaphoreType.DMA((E_LOC,)),
            pltpu.SemaphoreType.DMA((E_LOC,)),
            pltpu.SemaphoreType.DMA,
            pltpu.SemaphoreType.DMA,
        ],
        compiler_params=pltpu.CompilerParams(
            collective_id=0, vmem_limit_bytes=100 * 1024 * 1024),
    )(xb, asg[None, :], rank[None, :], W1, W2)
    return out.astype(jnp.float32)
